# Initial kernel scaffold; baseline (speedup 1.0000x reference)
import jax
import jax.numpy as jnp
from jax.experimental import pallas as pl


def kernel(x_cont, x_cat, tables, W1, b1, g1, beta1, W2, b2, g2, beta2, W3, b3):
    raise NotImplementedError("write your pallas kernel here")


# speedup vs baseline: 4.3803x; 4.3803x over previous
import functools

import jax
import jax.numpy as jnp
from jax import lax
from jax.experimental import pallas as pl
from jax.experimental.pallas import tpu as pltpu
from jax.experimental.pallas import tpu_sc as plsc

B = 16384
NFIELDS = 26
VOCAB = 100000
EDIM = 50
EDIMP = 64
CDIM = 13
H1 = 512
H2 = 256
EPS = 1e-5

NROWS = B * NFIELDS
NW = 32
ROWS_W = NROWS // NW
CHUNK = 128
NCHUNK = ROWS_W // CHUNK
GROUP = 4
NGROUP = NCHUNK // GROUP


def _gather_kernel(tab_hbm, idx_hbm, out_hbm, idx_v, rows_v, sem):
    wid = lax.axis_index("s") * 2 + lax.axis_index("c")
    base = wid * ROWS_W
    pltpu.sync_copy(idx_hbm.at[wid], idx_v)

    def body(g, _):
        cps = []
        for b in range(GROUP):
            j = g * GROUP + b
            cps.append(pltpu.async_copy(
                tab_hbm.at[idx_v.at[j]],
                rows_v.at[pl.ds(b * CHUNK, CHUNK)],
                sem,
            ))
        for cp in cps:
            cp.wait()
        pltpu.sync_copy(
            rows_v, out_hbm.at[pl.ds(base + g * GROUP * CHUNK, GROUP * CHUNK)])
        return 0

    lax.fori_loop(0, NGROUP, body, 0)


@functools.partial(jax.jit, donate_argnums=())
def _gather(tab2, idx3):
    mesh = plsc.VectorSubcoreMesh(core_axis_name="c", subcore_axis_name="s")
    k = pl.kernel(
        _gather_kernel,
        mesh=mesh,
        compiler_params=pltpu.CompilerParams(use_tc_tiling_on_sc=False),
        out_type=jax.ShapeDtypeStruct((NROWS, EDIMP), jnp.float32),
        scratch_types=[
            pltpu.VMEM((NCHUNK, CHUNK), jnp.int32),
            pltpu.VMEM((GROUP * CHUNK, EDIMP), jnp.float32),
            pltpu.SemaphoreType.DMA,
        ],
    )
    return k(tab2, idx3)


BM = 512
NB = B // BM


def _mlp_kernel(emb_ref, xc_ref, w1e_ref, w1c_ref, b1_ref, g1_ref, bt1_ref,
                w2_ref, b2_ref, g2_ref, bt2_ref, w3_ref, b3_ref,
                out_ref, h1_s, h2_s, s1, s2):
    p = pl.program_id(0)
    i = pl.program_id(1)
    dot = functools.partial(
        lax.dot_general,
        dimension_numbers=(((1,), (0,)), ((), ())),
        precision=lax.Precision.HIGHEST,
        preferred_element_type=jnp.float32,
    )

    @pl.when(jnp.logical_and(p == 0, i == 0))
    def _init():
        s1[...] = jnp.zeros_like(s1)
        s2[...] = jnp.zeros_like(s2)

    @pl.when(p == 0)
    def _pass0():
        h = dot(emb_ref[...], w1e_ref[...]) + dot(xc_ref[...], w1c_ref[...])
        h = h + b1_ref[...]
        h1_s[pl.ds(i * BM, BM), :] = h
        s1[0:1, :] = s1[0:1, :] + jnp.sum(h, axis=0, keepdims=True)
        s1[1:2, :] = s1[1:2, :] + jnp.sum(h * h, axis=0, keepdims=True)

    @pl.when(jnp.logical_and(p == 1, i == 0))
    def _stats1():
        mean = s1[0:1, :] * (1.0 / B)
        var = s1[1:2, :] * (1.0 / B) - mean * mean
        scale = g1_ref[...] * lax.rsqrt(var + EPS)
        s1[2:3, :] = scale
        s1[3:4, :] = bt1_ref[...] - mean * scale

    @pl.when(p == 1)
    def _pass1():
        h = h1_s[pl.ds(i * BM, BM), :]
        h = jnp.maximum(h * s1[2:3, :] + s1[3:4, :], 0.0)
        h2 = dot(h, w2_ref[...]) + b2_ref[...]
        h2_s[pl.ds(i * BM, BM), :] = h2
        s2[0:1, :] = s2[0:1, :] + jnp.sum(h2, axis=0, keepdims=True)
        s2[1:2, :] = s2[1:2, :] + jnp.sum(h2 * h2, axis=0, keepdims=True)

    @pl.when(jnp.logical_and(p == 2, i == 0))
    def _stats2():
        mean = s2[0:1, :] * (1.0 / B)
        var = s2[1:2, :] * (1.0 / B) - mean * mean
        scale = g2_ref[...] * lax.rsqrt(var + EPS)
        s2[2:3, :] = scale
        s2[3:4, :] = bt2_ref[...] - mean * scale

    @pl.when(p == 2)
    def _pass2():
        h = h2_s[pl.ds(i * BM, BM), :]
        h = jnp.maximum(h * s2[2:3, :] + s2[3:4, :], 0.0)
        logit = jnp.sum(h * w3_ref[...], axis=1, keepdims=True) + b3_ref[...]
        out_ref[...] = logit


def _mlp(emb, x_cont, W1e, W1c, b1, g1, beta1, W2, b2, g2, beta2, w3r, b3s):
    first = lambda p, i: (jnp.where(p == 0, i, 0), 0)
    fixed = lambda p, i: (0, 0)
    return pl.pallas_call(
        _mlp_kernel,
        grid=(3, NB),
        compiler_params=pltpu.CompilerParams(
            vmem_limit_bytes=100 * 1024 * 1024),
        in_specs=[
            pl.BlockSpec((BM, NFIELDS * EDIMP), first),
            pl.BlockSpec((BM, CDIM), first),
            pl.BlockSpec((NFIELDS * EDIMP, H1), fixed),
            pl.BlockSpec((CDIM, H1), fixed),
            pl.BlockSpec((1, H1), fixed),
            pl.BlockSpec((1, H1), fixed),
            pl.BlockSpec((1, H1), fixed),
            pl.BlockSpec((H1, H2), fixed),
            pl.BlockSpec((1, H2), fixed),
            pl.BlockSpec((1, H2), fixed),
            pl.BlockSpec((1, H2), fixed),
            pl.BlockSpec((1, H2), fixed),
            pl.BlockSpec((1, 1), fixed),
        ],
        out_specs=pl.BlockSpec((BM, 1), lambda p, i: (i, 0)),
        out_shape=jax.ShapeDtypeStruct((B, 1), jnp.float32),
        scratch_shapes=[
            pltpu.VMEM((B, H1), jnp.float32),
            pltpu.VMEM((B, H2), jnp.float32),
            pltpu.VMEM((8, H1), jnp.float32),
            pltpu.VMEM((8, H2), jnp.float32),
        ],
    )(emb, x_cont, W1e, W1c, b1, g1, beta1, W2, b2, g2, beta2, w3r, b3s)


def kernel(x_cont, x_cat, tables, W1, b1, g1, beta1, W2, b2, g2, beta2, W3, b3):
    tab2 = jnp.pad(tables.reshape(NFIELDS * VOCAB, EDIM),
                   ((0, 0), (0, EDIMP - EDIM)))
    offsets = (jnp.arange(NFIELDS, dtype=jnp.int32) * VOCAB)[None, :]
    idx3 = (x_cat + offsets).reshape(NW, NCHUNK, CHUNK)
    emb = _gather(tab2, idx3).reshape(B, NFIELDS * EDIMP)

    W1c = W1[:CDIM, :]
    W1e = jnp.pad(W1[CDIM:, :].reshape(NFIELDS, EDIM, H1),
                  ((0, 0), (0, EDIMP - EDIM), (0, 0))).reshape(NFIELDS * EDIMP, H1)
    return _mlp(
        emb, x_cont, W1e, W1c,
        b1.reshape(1, H1), g1.reshape(1, H1), beta1.reshape(1, H1),
        W2, b2.reshape(1, H2), g2.reshape(1, H2), beta2.reshape(1, H2),
        W3.reshape(1, H2), b3.reshape(1, 1),
    )


# speedup vs baseline: 4.6389x; 1.0590x over previous
import functools

import jax
import jax.numpy as jnp
from jax import lax
from jax.experimental import pallas as pl
from jax.experimental.pallas import tpu as pltpu
from jax.experimental.pallas import tpu_sc as plsc

B = 16384
NFIELDS = 26
VOCAB = 100000
EDIM = 50
EDIMP = 64
CDIM = 13
H1 = 512
H2 = 256
EPS = 1e-5

NROWS = B * NFIELDS
NW = 32
ROWS_W = NROWS // NW
CHUNK = 128
NCHUNK = ROWS_W // CHUNK
GROUP = 4
NGROUP = NCHUNK // GROUP


def _gather_kernel(tab_hbm, idx_hbm, out_hbm, idx_v, rows_v, sem):
    wid = lax.axis_index("s") * 2 + lax.axis_index("c")
    base = wid * ROWS_W
    pltpu.sync_copy(idx_hbm.at[wid], idx_v)

    def body(g, _):
        cps = []
        for b in range(GROUP):
            j = g * GROUP + b
            cps.append(pltpu.async_copy(
                tab_hbm.at[idx_v.at[j]],
                rows_v.at[pl.ds(b * CHUNK, CHUNK)],
                sem,
            ))
        for cp in cps:
            cp.wait()
        pltpu.sync_copy(
            rows_v, out_hbm.at[pl.ds(base + g * GROUP * CHUNK, GROUP * CHUNK)])
        return 0

    lax.fori_loop(0, NGROUP, body, 0)


@functools.partial(jax.jit, donate_argnums=())
def _gather(tab2, idx3):
    mesh = plsc.VectorSubcoreMesh(core_axis_name="c", subcore_axis_name="s")
    k = pl.kernel(
        _gather_kernel,
        mesh=mesh,
        compiler_params=pltpu.CompilerParams(use_tc_tiling_on_sc=False),
        out_type=jax.ShapeDtypeStruct((NROWS, EDIMP), jnp.float32),
        scratch_types=[
            pltpu.VMEM((NCHUNK, CHUNK), jnp.int32),
            pltpu.VMEM((GROUP * CHUNK, EDIMP), jnp.float32),
            pltpu.SemaphoreType.DMA,
        ],
    )
    return k(tab2, idx3)


PADR = 10000
NPAD = NFIELDS * VOCAB // PADR


def _pad_kernel(t_ref, o_ref):
    o_ref[:, :EDIM] = t_ref[...]
    o_ref[:, EDIM:] = jnp.zeros((PADR, EDIMP - EDIM), jnp.float32)


def _pad_table(tab):
    return pl.pallas_call(
        _pad_kernel,
        grid=(NPAD,),
        in_specs=[pl.BlockSpec((PADR, EDIM), lambda i: (i, 0))],
        out_specs=pl.BlockSpec((PADR, EDIMP), lambda i: (i, 0)),
        out_shape=jax.ShapeDtypeStruct((NFIELDS * VOCAB, EDIMP), jnp.float32),
    )(tab)


BM = 512
NB = B // BM


def _mlp_kernel(emb_ref, xc_ref, w1e_ref, w1c_ref, b1_ref, g1_ref, bt1_ref,
                w2_ref, b2_ref, g2_ref, bt2_ref, w3_ref, b3_ref,
                out_ref, h1_s, h2_s, s1, s2):
    p = pl.program_id(0)
    i = pl.program_id(1)
    dot = functools.partial(
        lax.dot_general,
        dimension_numbers=(((1,), (0,)), ((), ())),
        precision=lax.Precision.DEFAULT,
        preferred_element_type=jnp.float32,
    )

    @pl.when(jnp.logical_and(p == 0, i == 0))
    def _init():
        s1[...] = jnp.zeros_like(s1)
        s2[...] = jnp.zeros_like(s2)

    @pl.when(p == 0)
    def _pass0():
        h = dot(emb_ref[...], w1e_ref[...]) + dot(xc_ref[...], w1c_ref[...])
        h = h + b1_ref[...]
        h1_s[pl.ds(i * BM, BM), :] = h
        s1[0:1, :] = s1[0:1, :] + jnp.sum(h, axis=0, keepdims=True)
        s1[1:2, :] = s1[1:2, :] + jnp.sum(h * h, axis=0, keepdims=True)

    @pl.when(jnp.logical_and(p == 1, i == 0))
    def _stats1():
        mean = s1[0:1, :] * (1.0 / B)
        var = s1[1:2, :] * (1.0 / B) - mean * mean
        scale = g1_ref[...] * lax.rsqrt(var + EPS)
        s1[2:3, :] = scale
        s1[3:4, :] = bt1_ref[...] - mean * scale

    @pl.when(p == 1)
    def _pass1():
        h = h1_s[pl.ds(i * BM, BM), :]
        h = jnp.maximum(h * s1[2:3, :] + s1[3:4, :], 0.0)
        h2 = dot(h, w2_ref[...]) + b2_ref[...]
        h2_s[pl.ds(i * BM, BM), :] = h2
        s2[0:1, :] = s2[0:1, :] + jnp.sum(h2, axis=0, keepdims=True)
        s2[1:2, :] = s2[1:2, :] + jnp.sum(h2 * h2, axis=0, keepdims=True)

    @pl.when(jnp.logical_and(p == 2, i == 0))
    def _stats2():
        mean = s2[0:1, :] * (1.0 / B)
        var = s2[1:2, :] * (1.0 / B) - mean * mean
        scale = g2_ref[...] * lax.rsqrt(var + EPS)
        s2[2:3, :] = scale
        s2[3:4, :] = bt2_ref[...] - mean * scale

    @pl.when(p == 2)
    def _pass2():
        h = h2_s[pl.ds(i * BM, BM), :]
        h = jnp.maximum(h * s2[2:3, :] + s2[3:4, :], 0.0)
        logit = jnp.sum(h * w3_ref[...], axis=1, keepdims=True) + b3_ref[...]
        out_ref[...] = logit


def _mlp(emb, x_cont, W1e, W1c, b1, g1, beta1, W2, b2, g2, beta2, w3r, b3s):
    first = lambda p, i: (jnp.where(p == 0, i, 0), 0)
    fixed = lambda p, i: (0, 0)
    return pl.pallas_call(
        _mlp_kernel,
        grid=(3, NB),
        compiler_params=pltpu.CompilerParams(
            vmem_limit_bytes=100 * 1024 * 1024),
        in_specs=[
            pl.BlockSpec((BM, NFIELDS * EDIMP), first),
            pl.BlockSpec((BM, CDIM), first),
            pl.BlockSpec((NFIELDS * EDIMP, H1), fixed),
            pl.BlockSpec((CDIM, H1), fixed),
            pl.BlockSpec((1, H1), fixed),
            pl.BlockSpec((1, H1), fixed),
            pl.BlockSpec((1, H1), fixed),
            pl.BlockSpec((H1, H2), fixed),
            pl.BlockSpec((1, H2), fixed),
            pl.BlockSpec((1, H2), fixed),
            pl.BlockSpec((1, H2), fixed),
            pl.BlockSpec((1, H2), fixed),
            pl.BlockSpec((1, 1), fixed),
        ],
        out_specs=pl.BlockSpec((BM, 1), lambda p, i: (i, 0)),
        out_shape=jax.ShapeDtypeStruct((B, 1), jnp.float32),
        scratch_shapes=[
            pltpu.VMEM((B, H1), jnp.float32),
            pltpu.VMEM((B, H2), jnp.float32),
            pltpu.VMEM((8, H1), jnp.float32),
            pltpu.VMEM((8, H2), jnp.float32),
        ],
    )(emb, x_cont, W1e, W1c, b1, g1, beta1, W2, b2, g2, beta2, w3r, b3s)


def kernel(x_cont, x_cat, tables, W1, b1, g1, beta1, W2, b2, g2, beta2, W3, b3):
    tab2 = _pad_table(tables.reshape(NFIELDS * VOCAB, EDIM))
    offsets = (jnp.arange(NFIELDS, dtype=jnp.int32) * VOCAB)[None, :]
    idx3 = (x_cat + offsets).reshape(NW, NCHUNK, CHUNK)
    emb = _gather(tab2, idx3).reshape(B, NFIELDS * EDIMP)

    W1c = W1[:CDIM, :]
    W1e = jnp.pad(W1[CDIM:, :].reshape(NFIELDS, EDIM, H1),
                  ((0, 0), (0, EDIMP - EDIM), (0, 0))).reshape(NFIELDS * EDIMP, H1)
    return _mlp(
        emb, x_cont, W1e, W1c,
        b1.reshape(1, H1), g1.reshape(1, H1), beta1.reshape(1, H1),
        W2, b2.reshape(1, H2), g2.reshape(1, H2), beta2.reshape(1, H2),
        W3.reshape(1, H2), b3.reshape(1, 1),
    )
